# baseline (speedup 1.0000x reference)
import functools

import jax
import jax.numpy as jnp
from jax import lax
from jax.experimental import pallas as pl
from jax.experimental.pallas import tpu as pltpu
from jax.experimental.pallas import tpu_sc as plsc

NC = 2
NS = 16
LANES = 16
NPASS = 2

N = 10000
N2 = 10240
ACC = 5120
RTILE = 640
HTILE = 320

C = 128
SB = 4
NB = 12
CAP = NB * SB * C


def _sc_mesh():
    return plsc.VectorSubcoreMesh(
        core_axis_name="c", subcore_axis_name="s", num_cores=NC, num_subcores=NS
    )


@functools.lru_cache(maxsize=None)
def _make_agg_kernel():
    @functools.partial(
        pl.kernel,
        out_type=jax.ShapeDtypeStruct((NC, ACC, 128), jnp.float32),
        mesh=_sc_mesh(),
        scratch_types=[
            pltpu.VMEM((HTILE, 128), jnp.float32),
            pltpu.VMEM((SB, C), jnp.int32),
            pltpu.VMEM((SB, C), jnp.int32),
            pltpu.VMEM((SB, C), jnp.float32),
            pltpu.VMEM((2, C, 128), jnp.float32),
            pltpu.VMEM_SHARED((ACC, 128), jnp.float32),
            pltpu.SemaphoreType.DMA,
        ],
    )
    def agg_kernel(
        h_hbm, src_hbm, dst_hbm, ew_hbm, out_hbm,
        h_t, src_v, dst_v, ew_v, stage, acc_sh, sem_s,
    ):
        cid = lax.axis_index("c")
        sid = lax.axis_index("s")
        iota = lax.iota(jnp.int32, LANES)

        def _zrow(i, _):
            stage[0, i // 8, pl.ds((i % 8) * LANES, LANES)] = jnp.zeros(
                (LANES,), jnp.float32
            )
            return 0

        lax.fori_loop(0, C * 8, _zrow, 0)

        def _zcopy(k, _):
            pltpu.sync_copy(
                stage.at[0], acc_sh.at[pl.ds(sid * HTILE + k * C, C)]
            )
            return 0

        lax.fori_loop(0, HTILE // C + (1 if HTILE % C else 0) - 1, _zcopy, 0)
        pltpu.sync_copy(
            stage.at[0, pl.ds(0, 64)],
            acc_sh.at[pl.ds(sid * HTILE + 256, 64)],
        )
        plsc.subcore_barrier()

        def do_pass(p, _):
            pltpu.sync_copy(
                h_hbm.at[pl.ds(sid * RTILE + p * HTILE, HTILE)], h_t
            )

            def block(b, _):
                pltpu.sync_copy(src_hbm.at[cid, sid, p, b], src_v)
                pltpu.sync_copy(dst_hbm.at[cid, sid, p, b], dst_v)
                pltpu.sync_copy(ew_hbm.at[cid, sid, p, b], ew_v)

                def chunk(j, _):
                    cur = lax.rem(j, 2)

                    @pl.when(j >= 2)
                    def _():
                        pltpu.make_async_copy(
                            stage.at[cur], acc_sh.at[dst_v.at[j - 2]], sem_s
                        ).wait()

                    def group(g, _):
                        src16 = src_v[j, pl.ds(g * LANES, LANES)]
                        ew16 = ew_v[j, pl.ds(g * LANES, LANES)]
                        for l in range(LANES):
                            s = src16[l]
                            w = ew16[l]
                            e = g * LANES + l
                            for k in range(128 // LANES):
                                sl = pl.ds(k * LANES, LANES)
                                stage[cur, e, sl] = h_t[s, sl] * w
                        return 0

                    lax.fori_loop(0, C // LANES, group, 0)

                    pltpu.async_copy(
                        stage.at[cur], acc_sh.at[dst_v.at[j]], sem_s, add=True
                    )
                    return 0

                lax.fori_loop(0, SB, chunk, 0)
                pltpu.make_async_copy(
                    stage.at[0], acc_sh.at[dst_v.at[SB - 2]], sem_s
                ).wait()
                pltpu.make_async_copy(
                    stage.at[1], acc_sh.at[dst_v.at[SB - 1]], sem_s
                ).wait()
                return 0

            lax.fori_loop(0, NB, block, 0)
            return 0

        lax.fori_loop(0, NPASS, do_pass, 0)
        plsc.subcore_barrier()
        pltpu.sync_copy(
            acc_sh.at[pl.ds(sid * HTILE, HTILE)],
            out_hbm.at[cid].at[pl.ds(sid * HTILE, HTILE)],
        )

    return agg_kernel


def _bucket_edges(edge_index, edge_weight):
    e = edge_weight.shape[0]
    src = edge_index[0].astype(jnp.int32)
    dst = edge_index[1].astype(jnp.int32)
    ew = edge_weight.astype(jnp.float32)

    half = (dst >= ACC).astype(jnp.int32)
    tile = src // RTILE
    pas = (src % RTILE) // HTILE
    b = half * (2 * NS) + tile * 2 + pas
    v = b * (1 << 19) + jnp.arange(e, dtype=jnp.int32)
    sv = jnp.sort(v)
    ei = sv & ((1 << 19) - 1)
    starts = jnp.searchsorted(
        sv, jnp.arange(NC * NS * NPASS, dtype=jnp.int32) * (1 << 19)
    )
    ends = jnp.append(starts[1:], e)
    pidx = starts[:, None] + jnp.arange(CAP, dtype=jnp.int32)[None, :]
    valid = pidx < ends[:, None]
    eidx = ei[jnp.minimum(pidx, e - 1)]

    bb = jnp.arange(NC * NS * NPASS, dtype=jnp.int32)
    tl = (bb % (2 * NS)) // 2
    ps = bb % 2
    hf = bb // (2 * NS)
    src_l = src[eidx] - (tl * RTILE + ps * HTILE)[:, None]
    dst_l = dst[eidx] - (hf * ACC)[:, None]
    src_l = jnp.where(valid, src_l, 0)
    dst_l = jnp.where(valid, dst_l, 0)
    ew_b = jnp.where(valid, ew[eidx], 0.0)

    shp = (NC, NS, NPASS, NB, SB, C)
    return src_l.reshape(shp), dst_l.reshape(shp), ew_b.reshape(shp)


_ROWS = 320


def _half_spec():
    return pl.BlockSpec((1, _ROWS, 128), lambda i: (i // 16, i % 16, 0))


def _tc_first(p, dp, W, b):
    dh = W.shape[1]

    def body(p_ref, dp_ref, w_ref, b_ref, deg_ref, h_ref):
        deg = jnp.maximum(dp_ref[0, :, 0:1], 1e-6)
        deg_ref[...] = deg
        agg = p_ref[0] / deg
        h_ref[...] = jax.nn.gelu(
            jnp.dot(agg, w_ref[...], preferred_element_type=jnp.float32) + b_ref[...]
        )

    return pl.pallas_call(
        body,
        grid=(N2 // _ROWS,),
        in_specs=[
            _half_spec(),
            _half_spec(),
            pl.BlockSpec(W.shape, lambda i: (0, 0)),
            pl.BlockSpec((1, dh), lambda i: (0, 0)),
        ],
        out_specs=[
            pl.BlockSpec((_ROWS, 1), lambda i: (i, 0)),
            pl.BlockSpec((_ROWS, dh), lambda i: (i, 0)),
        ],
        out_shape=[
            jax.ShapeDtypeStruct((N2, 1), jnp.float32),
            jax.ShapeDtypeStruct((N2, dh), jnp.float32),
        ],
    )(p, dp, W, b)


def _tc_layer(p, deg, W, b):
    d = p.shape[2]
    dh = W.shape[1]

    def body(p_ref, deg_ref, w_ref, b_ref, aggn_ref, h_ref):
        agg = p_ref[0] / deg_ref[...]
        aggn_ref[...] = agg
        h_ref[...] = jax.nn.gelu(
            jnp.dot(agg, w_ref[...], preferred_element_type=jnp.float32) + b_ref[...]
        )

    return pl.pallas_call(
        body,
        grid=(N2 // _ROWS,),
        in_specs=[
            _half_spec(),
            pl.BlockSpec((_ROWS, 1), lambda i: (i, 0)),
            pl.BlockSpec(W.shape, lambda i: (0, 0)),
            pl.BlockSpec((1, dh), lambda i: (0, 0)),
        ],
        out_specs=[
            pl.BlockSpec((_ROWS, d), lambda i: (i, 0)),
            pl.BlockSpec((_ROWS, dh), lambda i: (i, 0)),
        ],
        out_shape=[
            jax.ShapeDtypeStruct((N2, d), jnp.float32),
            jax.ShapeDtypeStruct((N2, dh), jnp.float32),
        ],
    )(p, deg, W, b)


def _tc_dec(p, deg, skip_aggn, W_top, W_bot, b):
    d = p.shape[2]
    dh = W_top.shape[1]

    def body(p_ref, deg_ref, sk_ref, wt_ref, wb_ref, b_ref, h_ref):
        agg = p_ref[0] / deg_ref[...]
        acc = jnp.dot(agg, wt_ref[...], preferred_element_type=jnp.float32)
        acc = acc + jnp.dot(sk_ref[...], wb_ref[...], preferred_element_type=jnp.float32)
        h_ref[...] = jax.nn.gelu(acc + b_ref[...])

    return pl.pallas_call(
        body,
        grid=(N2 // _ROWS,),
        in_specs=[
            _half_spec(),
            pl.BlockSpec((_ROWS, 1), lambda i: (i, 0)),
            pl.BlockSpec((_ROWS, d), lambda i: (i, 0)),
            pl.BlockSpec(W_top.shape, lambda i: (0, 0)),
            pl.BlockSpec(W_bot.shape, lambda i: (0, 0)),
            pl.BlockSpec((1, dh), lambda i: (0, 0)),
        ],
        out_specs=pl.BlockSpec((_ROWS, dh), lambda i: (i, 0)),
        out_shape=jax.ShapeDtypeStruct((N2, dh), jnp.float32),
    )(p, deg, skip_aggn, W_top, W_bot, b)


def _tc_final(h, W, b):
    d = h.shape[1]
    do = W.shape[1]

    def body(h_ref, w_ref, b_ref, o_ref):
        o_ref[...] = (
            jnp.dot(h_ref[...], w_ref[...], preferred_element_type=jnp.float32)
            + b_ref[...]
        )

    return pl.pallas_call(
        body,
        grid=(N2 // _ROWS,),
        in_specs=[
            pl.BlockSpec((_ROWS, d), lambda i: (i, 0)),
            pl.BlockSpec(W.shape, lambda i: (0, 0)),
            pl.BlockSpec((1, do), lambda i: (0, 0)),
        ],
        out_specs=pl.BlockSpec((_ROWS, do), lambda i: (i, 0)),
        out_shape=jax.ShapeDtypeStruct((N2, do), jnp.float32),
    )(h, W, b)


def kernel(x, edge_index, edge_weight, W_in, b_in, W_enc, b_enc,
           W_bot, b_bot, W_dec, b_dec, W_out, b_out):
    src_b, dst_b, ew_b = _bucket_edges(edge_index, edge_weight)
    agg = _make_agg_kernel()

    ones = jnp.ones((N2, 128), jnp.float32)
    dp = agg(ones, src_b, dst_b, ew_b)

    x_pad = jnp.pad(x.astype(jnp.float32), ((0, N2 - N), (0, 0)))
    p = agg(x_pad, src_b, dst_b, ew_b)
    deg, h = _tc_first(p, dp, W_in, b_in.reshape(1, -1))

    n_layers = W_enc.shape[0]
    skip_aggs = {}
    for i in range(n_layers):
        p = agg(h, src_b, dst_b, ew_b)
        aggn, h = _tc_layer(p, deg, W_enc[i], b_enc[i].reshape(1, -1))
        if i >= 1:
            skip_aggs[i - 1] = aggn

    p = agg(h, src_b, dst_b, ew_b)
    aggn, h = _tc_layer(p, deg, W_bot, b_bot.reshape(1, -1))
    skip_aggs[n_layers - 1] = aggn

    d = x.shape[1]
    for i in range(n_layers):
        p = agg(h, src_b, dst_b, ew_b)
        h = _tc_dec(
            p, deg, skip_aggs[n_layers - 1 - i],
            W_dec[i][:d], W_dec[i][d:], b_dec[i].reshape(1, -1),
        )

    return _tc_final(h, W_out, b_out.reshape(1, -1))[:N]


# speedup vs baseline: 1.0723x; 1.0723x over previous
import functools

import jax
import jax.numpy as jnp
from jax import lax
from jax.experimental import pallas as pl
from jax.experimental.pallas import tpu as pltpu
from jax.experimental.pallas import tpu_sc as plsc

NC = 2
NS = 16
LANES = 16
NPASS = 2

N = 10000
N2 = 10240
ACC = 5120
RTILE = 640
HTILE = 320

C = 128
SB = 4
NB = 12
CAP = NB * SB * C


def _sc_mesh():
    return plsc.VectorSubcoreMesh(
        core_axis_name="c", subcore_axis_name="s", num_cores=NC, num_subcores=NS
    )


@functools.lru_cache(maxsize=None)
def _make_agg_kernel():
    @functools.partial(
        pl.kernel,
        out_type=jax.ShapeDtypeStruct((NC, ACC, 128), jnp.float32),
        mesh=_sc_mesh(),
        scratch_types=[
            pltpu.VMEM((HTILE, 128), jnp.float32),
            pltpu.VMEM((SB, C), jnp.int32),
            pltpu.VMEM((SB, C), jnp.int32),
            pltpu.VMEM((SB, C), jnp.float32),
            pltpu.VMEM((2, C, 128), jnp.float32),
            pltpu.VMEM((LANES,), jnp.int32),
            pltpu.VMEM_SHARED((ACC, 128), jnp.float32),
            pltpu.SemaphoreType.DMA,
        ],
    )
    def agg_kernel(
        h_hbm, src_hbm, dst_hbm, ew_hbm, nb_hbm, out_hbm,
        h_t, src_v, dst_v, ew_v, stage, nb_v, acc_sh, sem_s,
    ):
        cid = lax.axis_index("c")
        sid = lax.axis_index("s")
        iota = lax.iota(jnp.int32, LANES)

        def _zrow(i, _):
            stage[0, i // 8, pl.ds((i % 8) * LANES, LANES)] = jnp.zeros(
                (LANES,), jnp.float32
            )
            return 0

        lax.fori_loop(0, C * 8, _zrow, 0)

        def _zcopy(k, _):
            pltpu.sync_copy(
                stage.at[0], acc_sh.at[pl.ds(sid * HTILE + k * C, C)]
            )
            return 0

        lax.fori_loop(0, HTILE // C + (1 if HTILE % C else 0) - 1, _zcopy, 0)
        pltpu.sync_copy(
            stage.at[0, pl.ds(0, 64)],
            acc_sh.at[pl.ds(sid * HTILE + 256, 64)],
        )
        plsc.subcore_barrier()

        pltpu.sync_copy(nb_hbm.at[cid, sid], nb_v)
        nbv16 = nb_v[...]

        for p in range(NPASS):
            nblk = nbv16[p]
            pltpu.sync_copy(
                h_hbm.at[pl.ds(sid * RTILE + p * HTILE, HTILE)], h_t
            )

            def block(b, _):
                pltpu.sync_copy(src_hbm.at[cid, sid, p, b], src_v)
                pltpu.sync_copy(dst_hbm.at[cid, sid, p, b], dst_v)
                pltpu.sync_copy(ew_hbm.at[cid, sid, p, b], ew_v)

                def chunk(j, _):
                    cur = lax.rem(j, 2)

                    @pl.when(j >= 2)
                    def _():
                        pltpu.make_async_copy(
                            stage.at[cur], acc_sh.at[dst_v.at[j - 2]], sem_s
                        ).wait()

                    def group(g, _):
                        src16 = src_v[j, pl.ds(g * LANES, LANES)]
                        ew16 = ew_v[j, pl.ds(g * LANES, LANES)]
                        for l in range(LANES):
                            s = src16[l]
                            w = ew16[l]
                            e = g * LANES + l
                            for k in range(128 // LANES):
                                sl = pl.ds(k * LANES, LANES)
                                stage[cur, e, sl] = h_t[s, sl] * w
                        return 0

                    lax.fori_loop(0, C // LANES, group, 0)

                    pltpu.async_copy(
                        stage.at[cur], acc_sh.at[dst_v.at[j]], sem_s, add=True
                    )
                    return 0

                lax.fori_loop(0, SB, chunk, 0)
                pltpu.make_async_copy(
                    stage.at[0], acc_sh.at[dst_v.at[SB - 2]], sem_s
                ).wait()
                pltpu.make_async_copy(
                    stage.at[1], acc_sh.at[dst_v.at[SB - 1]], sem_s
                ).wait()
                return 0

            lax.fori_loop(0, nblk, block, 0)

        plsc.subcore_barrier()
        pltpu.sync_copy(
            acc_sh.at[pl.ds(sid * HTILE, HTILE)],
            out_hbm.at[cid].at[pl.ds(sid * HTILE, HTILE)],
        )

    return agg_kernel


def _bucket_edges(edge_index, edge_weight):
    e = edge_weight.shape[0]
    src = edge_index[0].astype(jnp.int32)
    dst = edge_index[1].astype(jnp.int32)
    ew = edge_weight.astype(jnp.float32)

    half = (dst >= ACC).astype(jnp.int32)
    tile = src // RTILE
    pas = (src % RTILE) // HTILE
    b = half * (2 * NS) + tile * 2 + pas
    v = b * (1 << 19) + jnp.arange(e, dtype=jnp.int32)
    sv = jnp.sort(v)
    ei = sv & ((1 << 19) - 1)
    starts = jnp.searchsorted(
        sv, jnp.arange(NC * NS * NPASS, dtype=jnp.int32) * (1 << 19)
    )
    ends = jnp.append(starts[1:], e)
    pidx = starts[:, None] + jnp.arange(CAP, dtype=jnp.int32)[None, :]
    valid = pidx < ends[:, None]
    eidx = ei[jnp.minimum(pidx, e - 1)]

    bb = jnp.arange(NC * NS * NPASS, dtype=jnp.int32)
    tl = (bb % (2 * NS)) // 2
    ps = bb % 2
    hf = bb // (2 * NS)
    src_l = src[eidx] - (tl * RTILE + ps * HTILE)[:, None]
    dst_l = dst[eidx] - (hf * ACC)[:, None]
    src_l = jnp.where(valid, src_l, 0)
    dst_l = jnp.where(valid, dst_l, 0)
    ew_b = jnp.where(valid, ew[eidx], 0.0)

    cnt = ends - starts
    nblk = jnp.minimum((cnt + SB * C - 1) // (SB * C), NB).astype(jnp.int32)
    nb_arr = jnp.zeros((NC, NS, LANES), jnp.int32)
    nb_arr = nb_arr.at[:, :, :NPASS].set(nblk.reshape(NC, NS, NPASS))

    shp = (NC, NS, NPASS, NB, SB, C)
    return src_l.reshape(shp), dst_l.reshape(shp), ew_b.reshape(shp), nb_arr


_ROWS = 320


def _half_spec():
    return pl.BlockSpec((1, _ROWS, 128), lambda i: (i // 16, i % 16, 0))


def _tc_first(p, dp, W, b):
    dh = W.shape[1]

    def body(p_ref, dp_ref, w_ref, b_ref, deg_ref, h_ref):
        deg = jnp.maximum(dp_ref[0, :, 0:1], 1e-6)
        deg_ref[...] = deg
        agg = p_ref[0] / deg
        h_ref[...] = jax.nn.gelu(
            jnp.dot(agg, w_ref[...], preferred_element_type=jnp.float32) + b_ref[...]
        )

    return pl.pallas_call(
        body,
        grid=(N2 // _ROWS,),
        in_specs=[
            _half_spec(),
            _half_spec(),
            pl.BlockSpec(W.shape, lambda i: (0, 0)),
            pl.BlockSpec((1, dh), lambda i: (0, 0)),
        ],
        out_specs=[
            pl.BlockSpec((_ROWS, 1), lambda i: (i, 0)),
            pl.BlockSpec((_ROWS, dh), lambda i: (i, 0)),
        ],
        out_shape=[
            jax.ShapeDtypeStruct((N2, 1), jnp.float32),
            jax.ShapeDtypeStruct((N2, dh), jnp.float32),
        ],
    )(p, dp, W, b)


def _tc_layer(p, deg, W, b):
    d = p.shape[2]
    dh = W.shape[1]

    def body(p_ref, deg_ref, w_ref, b_ref, aggn_ref, h_ref):
        agg = p_ref[0] / deg_ref[...]
        aggn_ref[...] = agg
        h_ref[...] = jax.nn.gelu(
            jnp.dot(agg, w_ref[...], preferred_element_type=jnp.float32) + b_ref[...]
        )

    return pl.pallas_call(
        body,
        grid=(N2 // _ROWS,),
        in_specs=[
            _half_spec(),
            pl.BlockSpec((_ROWS, 1), lambda i: (i, 0)),
            pl.BlockSpec(W.shape, lambda i: (0, 0)),
            pl.BlockSpec((1, dh), lambda i: (0, 0)),
        ],
        out_specs=[
            pl.BlockSpec((_ROWS, d), lambda i: (i, 0)),
            pl.BlockSpec((_ROWS, dh), lambda i: (i, 0)),
        ],
        out_shape=[
            jax.ShapeDtypeStruct((N2, d), jnp.float32),
            jax.ShapeDtypeStruct((N2, dh), jnp.float32),
        ],
    )(p, deg, W, b)


def _tc_dec(p, deg, skip_aggn, W_top, W_bot, b):
    d = p.shape[2]
    dh = W_top.shape[1]

    def body(p_ref, deg_ref, sk_ref, wt_ref, wb_ref, b_ref, h_ref):
        agg = p_ref[0] / deg_ref[...]
        acc = jnp.dot(agg, wt_ref[...], preferred_element_type=jnp.float32)
        acc = acc + jnp.dot(sk_ref[...], wb_ref[...], preferred_element_type=jnp.float32)
        h_ref[...] = jax.nn.gelu(acc + b_ref[...])

    return pl.pallas_call(
        body,
        grid=(N2 // _ROWS,),
        in_specs=[
            _half_spec(),
            pl.BlockSpec((_ROWS, 1), lambda i: (i, 0)),
            pl.BlockSpec((_ROWS, d), lambda i: (i, 0)),
            pl.BlockSpec(W_top.shape, lambda i: (0, 0)),
            pl.BlockSpec(W_bot.shape, lambda i: (0, 0)),
            pl.BlockSpec((1, dh), lambda i: (0, 0)),
        ],
        out_specs=pl.BlockSpec((_ROWS, dh), lambda i: (i, 0)),
        out_shape=jax.ShapeDtypeStruct((N2, dh), jnp.float32),
    )(p, deg, skip_aggn, W_top, W_bot, b)


def _tc_final(h, W, b):
    d = h.shape[1]
    do = W.shape[1]

    def body(h_ref, w_ref, b_ref, o_ref):
        o_ref[...] = (
            jnp.dot(h_ref[...], w_ref[...], preferred_element_type=jnp.float32)
            + b_ref[...]
        )

    return pl.pallas_call(
        body,
        grid=(N2 // _ROWS,),
        in_specs=[
            pl.BlockSpec((_ROWS, d), lambda i: (i, 0)),
            pl.BlockSpec(W.shape, lambda i: (0, 0)),
            pl.BlockSpec((1, do), lambda i: (0, 0)),
        ],
        out_specs=pl.BlockSpec((_ROWS, do), lambda i: (i, 0)),
        out_shape=jax.ShapeDtypeStruct((N2, do), jnp.float32),
    )(h, W, b)


def kernel(x, edge_index, edge_weight, W_in, b_in, W_enc, b_enc,
           W_bot, b_bot, W_dec, b_dec, W_out, b_out):
    src_b, dst_b, ew_b, nb_arr = _bucket_edges(edge_index, edge_weight)
    agg = _make_agg_kernel()

    ones = jnp.ones((N2, 128), jnp.float32)
    dp = agg(ones, src_b, dst_b, ew_b, nb_arr)

    x_pad = jnp.pad(x.astype(jnp.float32), ((0, N2 - N), (0, 0)))
    p = agg(x_pad, src_b, dst_b, ew_b, nb_arr)
    deg, h = _tc_first(p, dp, W_in, b_in.reshape(1, -1))

    n_layers = W_enc.shape[0]
    skip_aggs = {}
    for i in range(n_layers):
        p = agg(h, src_b, dst_b, ew_b, nb_arr)
        aggn, h = _tc_layer(p, deg, W_enc[i], b_enc[i].reshape(1, -1))
        if i >= 1:
            skip_aggs[i - 1] = aggn

    p = agg(h, src_b, dst_b, ew_b, nb_arr)
    aggn, h = _tc_layer(p, deg, W_bot, b_bot.reshape(1, -1))
    skip_aggs[n_layers - 1] = aggn

    d = x.shape[1]
    for i in range(n_layers):
        p = agg(h, src_b, dst_b, ew_b, nb_arr)
        h = _tc_dec(
            p, deg, skip_aggs[n_layers - 1 - i],
            W_dec[i][:d], W_dec[i][d:], b_dec[i].reshape(1, -1),
        )

    return _tc_final(h, W_out, b_out.reshape(1, -1))[:N]
